# baseline (speedup 1.0000x reference)
import jax
import jax.numpy as jnp
from jax.experimental import pallas as pl
from jax.experimental.pallas import tpu as pltpu


def _body(idx_ref, k_hbm, v_hbm, xk_hbm, xv_hbm, ok_hbm, ov_hbm, sem_k, sem_v, sem_r):
    ck = pltpu.make_async_copy(k_hbm, ok_hbm, sem_k)
    cv = pltpu.make_async_copy(v_hbm, ov_hbm, sem_v)
    ck.start()
    cv.start()
    ck.wait()
    cv.wait()
    idx = idx_ref[0]
    rk = pltpu.make_async_copy(xk_hbm, ok_hbm.at[:, :, pl.ds(idx, 1), :], sem_r)
    rv = pltpu.make_async_copy(xv_hbm, ov_hbm.at[:, :, pl.ds(idx, 1), :], sem_r)
    rk.start()
    rv.start()
    rk.wait()
    rv.wait()


def kernel(k_cache, v_cache, xk, xv, layer_past_len):
    idx = jnp.asarray(layer_past_len, jnp.int32).reshape((1,))
    grid_spec = pltpu.PrefetchScalarGridSpec(
        num_scalar_prefetch=1,
        grid=(1,),
        in_specs=[pl.BlockSpec(memory_space=pltpu.MemorySpace.HBM)] * 4,
        out_specs=[pl.BlockSpec(memory_space=pltpu.MemorySpace.HBM)] * 2,
        scratch_shapes=[pltpu.SemaphoreType.DMA] * 3,
    )
    return pl.pallas_call(
        _body,
        grid_spec=grid_spec,
        out_shape=(
            jax.ShapeDtypeStruct(k_cache.shape, k_cache.dtype),
            jax.ShapeDtypeStruct(v_cache.shape, v_cache.dtype),
        ),
    )(idx, k_cache, v_cache, xk, xv)
